# baseline (speedup 1.0000x reference)
import jax
import jax.numpy as jnp
from jax import lax
from jax.experimental import pallas as pl
from jax.experimental.pallas import tpu as pltpu
from jax.experimental.pallas import tpu_sc as plsc

N_NODES = 10000
N_EDGES = 160000
N_GRAPHS = 512
CLS = 10
H = 128
LOG_STD_MIN, LOG_STD_MAX = -5.0, 2.0

NC, NS = 2, 16
NW = NC * NS
NPW = 320
NPAD = NW * NPW
EW = N_EDGES // NW
CG = 200
MB = 256
BR = 512
BE = 2000
NEG_INF = float("-inf")


def _node_stage0_body(xt_ref, geo_ref, cat_ref, bat_ref, wall_ref, emb_ref,
                      catW_ref, catb_ref,
                      wW0_ref, wb0_ref, wW1_ref, wb1_ref,
                      gW0_ref, gb0_ref, gW1_ref, gb1_ref,
                      iW0_ref, ib0_ref, iW1_ref, ib1_ref,
                      mW0_ref, mb0_ref,
                      cond_ref, A_ref, B_ref):
    f32 = jnp.float32
    emb_t = jnp.tanh(emb_ref[...])
    cat_oh = (cat_ref[...] == lax.broadcasted_iota(jnp.int32, (BR, CLS), 1)).astype(f32)
    class_feat = jnp.tanh((cat_oh @ emb_t) @ catW_ref[...] + catb_ref[...])
    ws = jnp.tanh(wall_ref[...] @ wW0_ref[...] + wb0_ref[...]) @ wW1_ref[...] + wb1_ref[...]
    ws = jnp.tanh(ws)
    bat_oh = (bat_ref[...] == lax.broadcasted_iota(jnp.int32, (BR, N_GRAPHS), 1)).astype(f32)
    wall_feat = bat_oh @ ws
    geo_feat = jnp.tanh(jnp.tanh(geo_ref[...] @ gW0_ref[...] + gb0_ref[...]) @ gW1_ref[...] + gb1_ref[...])
    obj = jnp.tanh(jnp.tanh(xt_ref[...] @ iW0_ref[...] + ib0_ref[...]) @ iW1_ref[...] + ib1_ref[...])
    cond = jnp.concatenate([class_feat, wall_feat, geo_feat], axis=1)
    W0 = mW0_ref[...]
    Wd = W0[: H + 192] - W0[H + 192:]
    Wb = W0[H + 192:]
    h0 = jnp.concatenate([obj, cond], axis=1)
    cond_ref[...] = cond
    A_ref[...] = h0 @ Wd + mb0_ref[...]
    B_ref[...] = h0 @ Wb


def _node_mid_body(agg_ref, cond_ref, mW0_ref, mb0_ref, A_ref, B_ref):
    agg = agg_ref[...].astype(jnp.float32)
    hh = jnp.tanh(jnp.where(jnp.isfinite(agg), agg, 0.0))
    W0 = mW0_ref[...]
    Wd = W0[: H + 192] - W0[H + 192:]
    Wb = W0[H + 192:]
    h = jnp.concatenate([hh, cond_ref[...]], axis=1)
    A_ref[...] = h @ Wd + mb0_ref[...]
    B_ref[...] = h @ Wb


def _edge_body(Ad_ref, Bs_ref, W1_ref, b1_ref, msg_ref):
    msg_ref[...] = jnp.tanh(Ad_ref[...] + Bs_ref[...]) @ W1_ref[...] + b1_ref[...]


def _tail_body(agg_ref, cond_ref, tar_ref, tW0_ref, tb0_ref, tW1_ref, tb1_ref, out_ref):
    agg = agg_ref[...].astype(jnp.float32)
    hh = jnp.tanh(jnp.where(jnp.isfinite(agg), agg, 0.0))
    h = jnp.concatenate([hh, cond_ref[...]], axis=1)
    t = jnp.tanh(h @ tW0_ref[...] + tb0_ref[...])
    o = t @ tW1_ref[...] + tb1_ref[...]
    lane = lax.broadcasted_iota(jnp.int32, (BR, 8), 1)
    to = jnp.tanh(o)
    mu = to + jnp.tanh(tar_ref[...])
    ls = LOG_STD_MIN + 0.5 * (LOG_STD_MAX - LOG_STD_MIN) * (to + 1.0)
    out_ref[...] = jnp.where(lane < 3, mu, jnp.exp(ls))


def _whole(shape):
    return pl.BlockSpec(shape, lambda i: (0,) * len(shape))


def _rows(bshape):
    return pl.BlockSpec(bshape, lambda i: (i,) + (0,) * (len(bshape) - 1))


def _sc_gather_body(A_hbm, B_hbm, dst_hbm, src_hbm, Ad_hbm, Bs_hbm,
                    didx0, sidx0, rowsA0, rowsB0, didx1, sidx1, rowsA1, rowsB1,
                    semA0, semB0, semA1, semB1, semSA0, semSB0, semSA1, semSB1):
    wid = lax.axis_index("s") * NC + lax.axis_index("c")
    base = wid * EW
    bufs = [(didx0, sidx0, rowsA0, rowsB0, semA0, semB0, semSA0, semSB0),
            (didx1, sidx1, rowsA1, rowsB1, semA1, semB1, semSA1, semSB1)]
    nch = EW // CG
    store_pend = [None, None]

    def stage(i, p):
        didx, sidx, rowsA, rowsB, semA, semB, _, _ = bufs[p]
        if store_pend[p] is not None:
            sa, sb = store_pend[p]
            sa.wait()
            sb.wait()
            store_pend[p] = None
        b = base + i * CG
        pltpu.sync_copy(dst_hbm.at[pl.ds(b, CG)], didx)
        pltpu.sync_copy(src_hbm.at[pl.ds(b, CG)], sidx)
        ca = pltpu.async_copy(A_hbm.at[didx], rowsA, semA)
        cb = pltpu.async_copy(B_hbm.at[sidx], rowsB, semB)
        return ca, cb

    pend = stage(0, 0)
    for i in range(nch):
        p = i % 2
        nxt = stage(i + 1, (i + 1) % 2) if i + 1 < nch else None
        ca, cb = pend
        ca.wait()
        cb.wait()
        _, _, rowsA, rowsB, _, _, semSA, semSB = bufs[p]
        b = base + i * CG
        sa = pltpu.async_copy(rowsA, Ad_hbm.at[pl.ds(b, CG)], semSA)
        sb = pltpu.async_copy(rowsB, Bs_hbm.at[pl.ds(b, CG)], semSB)
        store_pend[p] = (sa, sb)
        pend = nxt

    for p in range(2):
        if store_pend[p] is not None:
            sa, sb = store_pend[p]
            sa.wait()
            sb.wait()


def _sc_gather(A, B, dstv, srcv):
    mesh = plsc.VectorSubcoreMesh(core_axis_name="c", subcore_axis_name="s",
                                  num_cores=NC, num_subcores=NS)
    f = pl.kernel(
        _sc_gather_body,
        out_type=(jax.ShapeDtypeStruct((N_EDGES, H), jnp.float32),
                  jax.ShapeDtypeStruct((N_EDGES, H), jnp.float32)),
        mesh=mesh,
        scratch_types=[
            pltpu.VMEM((CG,), jnp.int32),
            pltpu.VMEM((CG,), jnp.int32),
            pltpu.VMEM((CG, H), jnp.float32),
            pltpu.VMEM((CG, H), jnp.float32),
            pltpu.VMEM((CG,), jnp.int32),
            pltpu.VMEM((CG,), jnp.int32),
            pltpu.VMEM((CG, H), jnp.float32),
            pltpu.VMEM((CG, H), jnp.float32),
            pltpu.SemaphoreType.DMA,
            pltpu.SemaphoreType.DMA,
            pltpu.SemaphoreType.DMA,
            pltpu.SemaphoreType.DMA,
            pltpu.SemaphoreType.DMA,
            pltpu.SemaphoreType.DMA,
            pltpu.SemaphoreType.DMA,
            pltpu.SemaphoreType.DMA,
        ],
        compiler_params=pltpu.CompilerParams(needs_layout_passes=False),
    )
    return f(A, B, dstv, srcv)


EPW = 668 * MB
DHALF = N_EDGES // 2


def _sc_prep_body(dst_hbm, elist_hbm, llist_hbm, cnt_hbm, dbuf, ebuf, lbuf, cbuf, sm):
    wid = lax.axis_index("s") * NC + lax.axis_index("c")
    lo = wid * NPW

    zero16 = jnp.zeros((16,), jnp.int32)
    dummy16 = jnp.full((16,), NPW, jnp.int32)

    def init_l(k, c):
        lbuf[pl.ds(k * 16, 16)] = dummy16
        return c

    def init_e(k, c):
        ebuf[pl.ds(k * 16, 16)] = zero16
        return c

    lax.fori_loop(0, MB // 16, init_l, 0)
    lax.fori_loop(0, MB // 16, init_e, 0)
    sm[0] = 0

    def flush():
        f = sm[0]
        base = wid * EPW + f * MB
        pltpu.sync_copy(ebuf, elist_hbm.at[pl.ds(base, MB)])
        pltpu.sync_copy(lbuf, llist_hbm.at[pl.ds(base, MB)])
        lax.fori_loop(0, MB // 16, init_l, 0)
        sm[0] = f + 1

    iota16 = lax.iota(jnp.int32, 16)
    npw_u = jnp.uint32(NPW)

    cnt_vec = zero16
    for hhalf in range(2):
        pltpu.sync_copy(dst_hbm.at[pl.ds(hhalf * DHALF, DHALF)], dbuf)

        def group(g, cv, hhalf=hhalf):
            d16 = dbuf[pl.ds(g * 16, 16)]
            dl = d16 - lo
            m = plsc.bitcast(dl, jnp.uint32) < npw_u
            eid = hhalf * DHALF + g * 16 + iota16
            n = cv[0]
            plsc.store_compressed(ebuf.at[pl.ds(n, 16)], eid, mask=m)
            plsc.store_compressed(lbuf.at[pl.ds(n, 16)], dl, mask=m)
            cv = cv + plsc.all_reduce_population_count(m)
            fp = cv[0] > MB - 16

            @pl.when(fp)
            def _():
                flush()

            return jnp.where(fp, zero16, cv)

        cnt_vec = lax.fori_loop(0, DHALF // 16, group, cnt_vec)

    @pl.when(cnt_vec[0] > 0)
    def _():
        flush()

    cbuf[pl.ds(0, 16)] = zero16 + sm[0]
    pltpu.sync_copy(cbuf, cnt_hbm.at[pl.ds(wid * 16, 16)])


def _sc_prep(dstv):
    mesh = plsc.VectorSubcoreMesh(core_axis_name="c", subcore_axis_name="s",
                                  num_cores=NC, num_subcores=NS)
    f = pl.kernel(
        _sc_prep_body,
        out_type=(jax.ShapeDtypeStruct((NW * EPW,), jnp.int32),
                  jax.ShapeDtypeStruct((NW * EPW,), jnp.int32),
                  jax.ShapeDtypeStruct((NW * 16,), jnp.int32)),
        mesh=mesh,
        scratch_types=[
            pltpu.VMEM((DHALF,), jnp.int32),
            pltpu.VMEM((MB,), jnp.int32),
            pltpu.VMEM((MB,), jnp.int32),
            pltpu.VMEM((16,), jnp.int32),
            pltpu.SMEM((2,), jnp.int32),
        ],
        compiler_params=pltpu.CompilerParams(needs_layout_passes=False),
    )
    return f(dstv)


def _sc_scatter_body(msg_hbm, elist_hbm, llist_hbm, cnt_hbm, agg_hbm,
                     acc, ebuf0, lbuf0, rows0, ebuf1, lbuf1, rows1, cbuf, sem0, sem1):
    wid = lax.axis_index("s") * NC + lax.axis_index("c")
    lo = wid * NPW
    neg = jnp.full((16,), NEG_INF, jnp.float32)

    def init_acc(k, c):
        acc[pl.ds(k * 16, 16)] = neg
        return c

    lax.fori_loop(0, (NPW + 1) * H // 16, init_acc, 0)

    pltpu.sync_copy(cnt_hbm.at[pl.ds(wid * 16, 16)], cbuf)
    nf = cbuf[pl.ds(0, 16)][0]

    sets = [(ebuf0, lbuf0, rows0, sem0), (ebuf1, lbuf1, rows1, sem1)]

    def start(f, s):
        ebuf, lbuf, rows, sem = s
        base = wid * EPW + f * MB
        pltpu.sync_copy(elist_hbm.at[pl.ds(base, MB)], ebuf)
        pltpu.sync_copy(llist_hbm.at[pl.ds(base, MB)], lbuf)
        pltpu.async_copy(msg_hbm.at[ebuf], rows, sem)

    def process(s):
        ebuf, lbuf, rows, sem = s
        pltpu.make_async_copy(msg_hbm.at[ebuf], rows, sem).wait()

        def one16(q, c2):
            dls = lbuf[pl.ds(q * 16, 16)]
            bases = [dls[t] * H for t in range(16)]
            for t in range(16):
                i = q * 16 + t
                for j in range(H // 16):
                    a = acc[pl.ds(bases[t] + j * 16, 16)]
                    r = rows[i, pl.ds(j * 16, 16)]
                    acc[pl.ds(bases[t] + j * 16, 16)] = jnp.maximum(a, r)
            return c2

        lax.fori_loop(0, MB // 16, one16, 0)

    @pl.when(nf > 0)
    def _():
        start(0, sets[0])

    def body(f, c):
        for par in range(2):
            @pl.when(f % 2 == par)
            def _(par=par):
                @pl.when(f + 1 < nf)
                def _():
                    start(f + 1, sets[1 - par])

                process(sets[par])

        return c

    lax.fori_loop(0, nf, body, 0)

    pltpu.sync_copy(acc.at[pl.ds(0, NPW * H)], agg_hbm.at[pl.ds(lo * H, NPW * H)])


def _sc_scatter(msg, elist, llist, counts):
    mesh = plsc.VectorSubcoreMesh(core_axis_name="c", subcore_axis_name="s",
                                  num_cores=NC, num_subcores=NS)
    f = pl.kernel(
        _sc_scatter_body,
        out_type=jax.ShapeDtypeStruct((NPAD * H,), jnp.float32),
        mesh=mesh,
        scratch_types=[
            pltpu.VMEM(((NPW + 1) * H,), jnp.float32),
            pltpu.VMEM((MB,), jnp.int32),
            pltpu.VMEM((MB,), jnp.int32),
            pltpu.VMEM((MB, H), jnp.float32),
            pltpu.VMEM((MB,), jnp.int32),
            pltpu.VMEM((MB,), jnp.int32),
            pltpu.VMEM((MB, H), jnp.float32),
            pltpu.VMEM((16,), jnp.int32),
            pltpu.SemaphoreType.DMA,
            pltpu.SemaphoreType.DMA,
        ],
        compiler_params=pltpu.CompilerParams(needs_layout_passes=False),
    )
    return f(msg, elist, llist, counts)


def kernel(x, geo, wall_batch, tar_scores, emb_table, cat_W, cat_b, wall_W0, wall_b0, wall_W1, wall_b1, geo_W0, geo_b0, geo_W1, geo_b1, init_W0, init_b0, init_W1, init_b1, m1_W0, m1_b0, m1_W1, m1_b1, m2_W0, m2_b0, m2_W1, m2_b1, tail_W0, tail_b0, tail_W1, tail_b1, category, edge_index, batch):
    f32 = jnp.float32

    def padn(a):
        return jnp.pad(a, ((0, NPAD - N_NODES), (0, 0)))

    def r2(b):
        return b[None, :]

    xt = jnp.pad(padn(jnp.concatenate([x, tar_scores], axis=1)), ((0, 0), (0, 1)))
    iW0 = jnp.pad(init_W0, ((0, 1), (0, 0)))
    geo_p = padn(geo)
    cat_p = padn(category)
    bat_p = padn(batch[:, None])
    tar8 = jnp.pad(padn(tar_scores), ((0, 0), (0, 5)))
    tW1 = jnp.pad(tail_W1, ((0, 0), (0, 2)))
    tb1 = jnp.pad(tail_b1, (0, 2))[None, :]
    srcv = edge_index[0]
    dstv = edge_index[1]

    gridn = (NPAD // BR,)
    cond, A1, B1 = pl.pallas_call(
        _node_stage0_body,
        grid=gridn,
        in_specs=[
            _rows((BR, 8)), _rows((BR, 2)), _rows((BR, 1)), _rows((BR, 1)),
            _whole((N_GRAPHS, 1)), _whole((CLS, 64)),
            _whole((64, 64)), _whole((1, 64)),
            _whole((1, 64)), _whole((1, 64)), _whole((64, 64)), _whole((1, 64)),
            _whole((2, 64)), _whole((1, 64)), _whole((64, 64)), _whole((1, 64)),
            _whole((8, H)), _whole((1, H)), _whole((H, H)), _whole((1, H)),
            _whole((2 * (H + 192), H)), _whole((1, H)),
        ],
        out_specs=[_rows((BR, 192)), _rows((BR, H)), _rows((BR, H))],
        out_shape=[
            jax.ShapeDtypeStruct((NPAD, 192), f32),
            jax.ShapeDtypeStruct((NPAD, H), f32),
            jax.ShapeDtypeStruct((NPAD, H), f32),
        ],
    )(xt, geo_p, cat_p, bat_p, wall_batch, emb_table,
      cat_W, r2(cat_b), wall_W0, r2(wall_b0), wall_W1, r2(wall_b1),
      geo_W0, r2(geo_b0), geo_W1, r2(geo_b1), iW0, r2(init_b0), init_W1, r2(init_b1),
      m1_W0, r2(m1_b0))

    gride = (N_EDGES // BE,)

    def edge_call(Ad, Bs, W1, b1):
        return pl.pallas_call(
            _edge_body,
            grid=gride,
            in_specs=[_rows((BE, H)), _rows((BE, H)), _whole((H, H)), _whole((1, H))],
            out_specs=_rows((BE, H)),
            out_shape=jax.ShapeDtypeStruct((N_EDGES, H), f32),
        )(Ad, Bs, W1, b1)

    def mid_call(agg, cond, mW0, mb0):
        return pl.pallas_call(
            _node_mid_body,
            grid=gridn,
            in_specs=[_rows((BR, H)), _rows((BR, 192)),
                      _whole((2 * (H + 192), H)), _whole((1, H))],
            out_specs=[_rows((BR, H)), _rows((BR, H))],
            out_shape=[jax.ShapeDtypeStruct((NPAD, H), f32),
                       jax.ShapeDtypeStruct((NPAD, H), f32)],
        )(agg, cond, mW0, mb0)

    elist, llist, counts = _sc_prep(dstv)
    Ad, Bs = _sc_gather(A1, B1, dstv, srcv)
    msg1 = edge_call(Ad, Bs, m1_W1, r2(m1_b1))
    agg1 = _sc_scatter(msg1, elist, llist, counts).reshape(NPAD, H)
    A2, B2 = mid_call(agg1, cond, m2_W0, r2(m2_b0))
    Ad2, Bs2 = _sc_gather(A2, B2, dstv, srcv)
    msg2 = edge_call(Ad2, Bs2, m2_W1, r2(m2_b1))
    agg2 = _sc_scatter(msg2, elist, llist, counts).reshape(NPAD, H)

    out8 = pl.pallas_call(
        _tail_body,
        grid=gridn,
        in_specs=[_rows((BR, H)), _rows((BR, 192)), _rows((BR, 8)),
                  _whole((H + 192, H)), _whole((1, H)), _whole((H, 8)), _whole((1, 8))],
        out_specs=_rows((BR, 8)),
        out_shape=jax.ShapeDtypeStruct((NPAD, 8), f32),
    )(agg2, cond, tar8, tail_W0, r2(tail_b0), tW1, tb1)

    return out8[:N_NODES, :6]


# speedup vs baseline: 1.0035x; 1.0035x over previous
import jax
import jax.numpy as jnp
from jax import lax
from jax.experimental import pallas as pl
from jax.experimental.pallas import tpu as pltpu
from jax.experimental.pallas import tpu_sc as plsc

N_NODES = 10000
N_EDGES = 160000
N_GRAPHS = 512
CLS = 10
H = 128
LOG_STD_MIN, LOG_STD_MAX = -5.0, 2.0

NC, NS = 2, 16
NW = NC * NS
NPW = 320
NPAD = NW * NPW
EW = N_EDGES // NW
CG = 200
MB = 256
BR = 512
BE = 2000
NEG_INF = float("-inf")


def _node_stage0_body(xt_ref, geo_ref, cat_ref, bat_ref, wall_ref, emb_ref,
                      catW_ref, catb_ref,
                      wW0_ref, wb0_ref, wW1_ref, wb1_ref,
                      gW0_ref, gb0_ref, gW1_ref, gb1_ref,
                      iW0_ref, ib0_ref, iW1_ref, ib1_ref,
                      mW0_ref, mb0_ref,
                      cond_ref, A_ref, B_ref):
    f32 = jnp.float32
    emb_t = jnp.tanh(emb_ref[...])
    cat_oh = (cat_ref[...] == lax.broadcasted_iota(jnp.int32, (BR, CLS), 1)).astype(f32)
    class_feat = jnp.tanh((cat_oh @ emb_t) @ catW_ref[...] + catb_ref[...])
    ws = jnp.tanh(wall_ref[...] @ wW0_ref[...] + wb0_ref[...]) @ wW1_ref[...] + wb1_ref[...]
    ws = jnp.tanh(ws)
    bat_oh = (bat_ref[...] == lax.broadcasted_iota(jnp.int32, (BR, N_GRAPHS), 1)).astype(f32)
    wall_feat = bat_oh @ ws
    geo_feat = jnp.tanh(jnp.tanh(geo_ref[...] @ gW0_ref[...] + gb0_ref[...]) @ gW1_ref[...] + gb1_ref[...])
    obj = jnp.tanh(jnp.tanh(xt_ref[...] @ iW0_ref[...] + ib0_ref[...]) @ iW1_ref[...] + ib1_ref[...])
    cond = jnp.concatenate([class_feat, wall_feat, geo_feat], axis=1)
    W0 = mW0_ref[...]
    Wd = W0[: H + 192] - W0[H + 192:]
    Wb = W0[H + 192:]
    h0 = jnp.concatenate([obj, cond], axis=1)
    cond_ref[...] = cond
    A_ref[...] = h0 @ Wd + mb0_ref[...]
    B_ref[...] = h0 @ Wb


def _node_mid_body(agg_ref, cond_ref, mW0_ref, mb0_ref, A_ref, B_ref):
    agg = agg_ref[...].astype(jnp.float32)
    hh = jnp.tanh(jnp.where(jnp.isfinite(agg), agg, 0.0))
    W0 = mW0_ref[...]
    Wd = W0[: H + 192] - W0[H + 192:]
    Wb = W0[H + 192:]
    h = jnp.concatenate([hh, cond_ref[...]], axis=1)
    A_ref[...] = h @ Wd + mb0_ref[...]
    B_ref[...] = h @ Wb


def _edge_body(Ad_ref, Bs_ref, W1_ref, b1_ref, msg_ref):
    msg_ref[...] = jnp.tanh(Ad_ref[...] + Bs_ref[...]) @ W1_ref[...] + b1_ref[...]


def _tail_body(agg_ref, cond_ref, tar_ref, tW0_ref, tb0_ref, tW1_ref, tb1_ref, out_ref):
    agg = agg_ref[...].astype(jnp.float32)
    hh = jnp.tanh(jnp.where(jnp.isfinite(agg), agg, 0.0))
    h = jnp.concatenate([hh, cond_ref[...]], axis=1)
    t = jnp.tanh(h @ tW0_ref[...] + tb0_ref[...])
    o = t @ tW1_ref[...] + tb1_ref[...]
    lane = lax.broadcasted_iota(jnp.int32, (BR, 8), 1)
    to = jnp.tanh(o)
    mu = to + jnp.tanh(tar_ref[...])
    ls = LOG_STD_MIN + 0.5 * (LOG_STD_MAX - LOG_STD_MIN) * (to + 1.0)
    out_ref[...] = jnp.where(lane < 3, mu, jnp.exp(ls))


def _whole(shape):
    return pl.BlockSpec(shape, lambda i: (0,) * len(shape))


def _rows(bshape):
    return pl.BlockSpec(bshape, lambda i: (i,) + (0,) * (len(bshape) - 1))


def _sc_gather_body(A_hbm, B_hbm, dst_hbm, src_hbm, Ad_hbm, Bs_hbm,
                    didx0, sidx0, rowsA0, rowsB0, didx1, sidx1, rowsA1, rowsB1,
                    semA0, semB0, semA1, semB1, semSA0, semSB0, semSA1, semSB1):
    wid = lax.axis_index("s") * NC + lax.axis_index("c")
    base = wid * EW
    bufs = [(didx0, sidx0, rowsA0, rowsB0, semA0, semB0, semSA0, semSB0),
            (didx1, sidx1, rowsA1, rowsB1, semA1, semB1, semSA1, semSB1)]
    nch = EW // CG
    store_pend = [None, None]

    def stage(i, p):
        didx, sidx, rowsA, rowsB, semA, semB, _, _ = bufs[p]
        if store_pend[p] is not None:
            sa, sb = store_pend[p]
            sa.wait()
            sb.wait()
            store_pend[p] = None
        b = base + i * CG
        pltpu.sync_copy(dst_hbm.at[pl.ds(b, CG)], didx)
        pltpu.sync_copy(src_hbm.at[pl.ds(b, CG)], sidx)
        ca = pltpu.async_copy(A_hbm.at[didx], rowsA, semA)
        cb = pltpu.async_copy(B_hbm.at[sidx], rowsB, semB)
        return ca, cb

    pend = stage(0, 0)
    for i in range(nch):
        p = i % 2
        nxt = stage(i + 1, (i + 1) % 2) if i + 1 < nch else None
        ca, cb = pend
        ca.wait()
        cb.wait()
        _, _, rowsA, rowsB, _, _, semSA, semSB = bufs[p]
        b = base + i * CG
        sa = pltpu.async_copy(rowsA, Ad_hbm.at[pl.ds(b, CG)], semSA)
        sb = pltpu.async_copy(rowsB, Bs_hbm.at[pl.ds(b, CG)], semSB)
        store_pend[p] = (sa, sb)
        pend = nxt

    for p in range(2):
        if store_pend[p] is not None:
            sa, sb = store_pend[p]
            sa.wait()
            sb.wait()


def _sc_gather(A, B, dstv, srcv):
    mesh = plsc.VectorSubcoreMesh(core_axis_name="c", subcore_axis_name="s",
                                  num_cores=NC, num_subcores=NS)
    f = pl.kernel(
        _sc_gather_body,
        out_type=(jax.ShapeDtypeStruct((N_EDGES, H), jnp.float32),
                  jax.ShapeDtypeStruct((N_EDGES, H), jnp.float32)),
        mesh=mesh,
        scratch_types=[
            pltpu.VMEM((CG,), jnp.int32),
            pltpu.VMEM((CG,), jnp.int32),
            pltpu.VMEM((CG, H), jnp.float32),
            pltpu.VMEM((CG, H), jnp.float32),
            pltpu.VMEM((CG,), jnp.int32),
            pltpu.VMEM((CG,), jnp.int32),
            pltpu.VMEM((CG, H), jnp.float32),
            pltpu.VMEM((CG, H), jnp.float32),
            pltpu.SemaphoreType.DMA,
            pltpu.SemaphoreType.DMA,
            pltpu.SemaphoreType.DMA,
            pltpu.SemaphoreType.DMA,
            pltpu.SemaphoreType.DMA,
            pltpu.SemaphoreType.DMA,
            pltpu.SemaphoreType.DMA,
            pltpu.SemaphoreType.DMA,
        ],
        compiler_params=pltpu.CompilerParams(needs_layout_passes=False),
    )
    return f(A, B, dstv, srcv)


EPW = 668 * MB
DHALF = N_EDGES // 2


def _sc_prep_body(dst_hbm, elist_hbm, llist_hbm, cnt_hbm, dbuf, ebuf, lbuf, cbuf, sm):
    wid = lax.axis_index("s") * NC + lax.axis_index("c")
    lo = wid * NPW

    zero16 = jnp.zeros((16,), jnp.int32)
    dummy16 = jnp.full((16,), NPW, jnp.int32)

    def init_l(k, c):
        lbuf[pl.ds(k * 16, 16)] = dummy16
        return c

    def init_e(k, c):
        ebuf[pl.ds(k * 16, 16)] = zero16
        return c

    lax.fori_loop(0, MB // 16, init_l, 0)
    lax.fori_loop(0, MB // 16, init_e, 0)
    sm[0] = 0

    def flush():
        f = sm[0]
        base = wid * EPW + f * MB
        pltpu.sync_copy(ebuf, elist_hbm.at[pl.ds(base, MB)])
        pltpu.sync_copy(lbuf, llist_hbm.at[pl.ds(base, MB)])
        lax.fori_loop(0, MB // 16, init_l, 0)
        sm[0] = f + 1

    iota16 = lax.iota(jnp.int32, 16)
    npw_u = jnp.uint32(NPW)

    cnt_vec = zero16
    for hhalf in range(2):
        pltpu.sync_copy(dst_hbm.at[pl.ds(hhalf * DHALF, DHALF)], dbuf)

        def group(g, cv, hhalf=hhalf):
            d16 = dbuf[pl.ds(g * 16, 16)]
            dl = d16 - lo
            m = plsc.bitcast(dl, jnp.uint32) < npw_u
            eid = hhalf * DHALF + g * 16 + iota16
            n = cv[0]
            plsc.store_compressed(ebuf.at[pl.ds(n, 16)], eid, mask=m)
            plsc.store_compressed(lbuf.at[pl.ds(n, 16)], dl, mask=m)
            cv = cv + plsc.all_reduce_population_count(m)
            fp = cv[0] > MB - 16

            @pl.when(fp)
            def _():
                flush()

            return jnp.where(fp, zero16, cv)

        def group4(q, cv):
            for u in range(4):
                cv = group(q * 4 + u, cv)
            return cv

        cnt_vec = lax.fori_loop(0, DHALF // 64, group4, cnt_vec)

    @pl.when(cnt_vec[0] > 0)
    def _():
        flush()

    cbuf[pl.ds(0, 16)] = zero16 + sm[0]
    pltpu.sync_copy(cbuf, cnt_hbm.at[pl.ds(wid * 16, 16)])


def _sc_prep(dstv):
    mesh = plsc.VectorSubcoreMesh(core_axis_name="c", subcore_axis_name="s",
                                  num_cores=NC, num_subcores=NS)
    f = pl.kernel(
        _sc_prep_body,
        out_type=(jax.ShapeDtypeStruct((NW * EPW,), jnp.int32),
                  jax.ShapeDtypeStruct((NW * EPW,), jnp.int32),
                  jax.ShapeDtypeStruct((NW * 16,), jnp.int32)),
        mesh=mesh,
        scratch_types=[
            pltpu.VMEM((DHALF,), jnp.int32),
            pltpu.VMEM((MB,), jnp.int32),
            pltpu.VMEM((MB,), jnp.int32),
            pltpu.VMEM((16,), jnp.int32),
            pltpu.SMEM((2,), jnp.int32),
        ],
        compiler_params=pltpu.CompilerParams(needs_layout_passes=False),
    )
    return f(dstv)


def _sc_scatter_body(msg_hbm, elist_hbm, llist_hbm, cnt_hbm, agg_hbm,
                     acc, ebuf0, lbuf0, rows0, ebuf1, lbuf1, rows1, cbuf, sem0, sem1):
    wid = lax.axis_index("s") * NC + lax.axis_index("c")
    lo = wid * NPW
    neg = jnp.full((16,), NEG_INF, jnp.float32)

    def init_acc(k, c):
        acc[pl.ds(k * 16, 16)] = neg
        return c

    lax.fori_loop(0, (NPW + 1) * H // 16, init_acc, 0)

    pltpu.sync_copy(cnt_hbm.at[pl.ds(wid * 16, 16)], cbuf)
    nf = cbuf[pl.ds(0, 16)][0]

    sets = [(ebuf0, lbuf0, rows0, sem0), (ebuf1, lbuf1, rows1, sem1)]

    def start(f, s):
        ebuf, lbuf, rows, sem = s
        base = wid * EPW + f * MB
        pltpu.sync_copy(elist_hbm.at[pl.ds(base, MB)], ebuf)
        pltpu.sync_copy(llist_hbm.at[pl.ds(base, MB)], lbuf)
        pltpu.async_copy(msg_hbm.at[ebuf], rows, sem)

    def process(s):
        ebuf, lbuf, rows, sem = s
        pltpu.make_async_copy(msg_hbm.at[ebuf], rows, sem).wait()

        def one16(q, c2):
            dls = lbuf[pl.ds(q * 16, 16)]
            bases = [dls[t] * H for t in range(16)]
            for t in range(16):
                i = q * 16 + t
                for j in range(H // 16):
                    a = acc[pl.ds(bases[t] + j * 16, 16)]
                    r = rows[i, pl.ds(j * 16, 16)]
                    acc[pl.ds(bases[t] + j * 16, 16)] = jnp.maximum(a, r)
            return c2

        lax.fori_loop(0, MB // 16, one16, 0)

    @pl.when(nf > 0)
    def _():
        start(0, sets[0])

    def body(f, c):
        for par in range(2):
            @pl.when(f % 2 == par)
            def _(par=par):
                @pl.when(f + 1 < nf)
                def _():
                    start(f + 1, sets[1 - par])

                process(sets[par])

        return c

    lax.fori_loop(0, nf, body, 0)

    pltpu.sync_copy(acc.at[pl.ds(0, NPW * H)], agg_hbm.at[pl.ds(lo * H, NPW * H)])


def _sc_scatter(msg, elist, llist, counts):
    mesh = plsc.VectorSubcoreMesh(core_axis_name="c", subcore_axis_name="s",
                                  num_cores=NC, num_subcores=NS)
    f = pl.kernel(
        _sc_scatter_body,
        out_type=jax.ShapeDtypeStruct((NPAD * H,), jnp.float32),
        mesh=mesh,
        scratch_types=[
            pltpu.VMEM(((NPW + 1) * H,), jnp.float32),
            pltpu.VMEM((MB,), jnp.int32),
            pltpu.VMEM((MB,), jnp.int32),
            pltpu.VMEM((MB, H), jnp.float32),
            pltpu.VMEM((MB,), jnp.int32),
            pltpu.VMEM((MB,), jnp.int32),
            pltpu.VMEM((MB, H), jnp.float32),
            pltpu.VMEM((16,), jnp.int32),
            pltpu.SemaphoreType.DMA,
            pltpu.SemaphoreType.DMA,
        ],
        compiler_params=pltpu.CompilerParams(needs_layout_passes=False),
    )
    return f(msg, elist, llist, counts)


def kernel(x, geo, wall_batch, tar_scores, emb_table, cat_W, cat_b, wall_W0, wall_b0, wall_W1, wall_b1, geo_W0, geo_b0, geo_W1, geo_b1, init_W0, init_b0, init_W1, init_b1, m1_W0, m1_b0, m1_W1, m1_b1, m2_W0, m2_b0, m2_W1, m2_b1, tail_W0, tail_b0, tail_W1, tail_b1, category, edge_index, batch):
    f32 = jnp.float32

    def padn(a):
        return jnp.pad(a, ((0, NPAD - N_NODES), (0, 0)))

    def r2(b):
        return b[None, :]

    xt = jnp.pad(padn(jnp.concatenate([x, tar_scores], axis=1)), ((0, 0), (0, 1)))
    iW0 = jnp.pad(init_W0, ((0, 1), (0, 0)))
    geo_p = padn(geo)
    cat_p = padn(category)
    bat_p = padn(batch[:, None])
    tar8 = jnp.pad(padn(tar_scores), ((0, 0), (0, 5)))
    tW1 = jnp.pad(tail_W1, ((0, 0), (0, 2)))
    tb1 = jnp.pad(tail_b1, (0, 2))[None, :]
    srcv = edge_index[0]
    dstv = edge_index[1]

    gridn = (NPAD // BR,)
    cond, A1, B1 = pl.pallas_call(
        _node_stage0_body,
        grid=gridn,
        in_specs=[
            _rows((BR, 8)), _rows((BR, 2)), _rows((BR, 1)), _rows((BR, 1)),
            _whole((N_GRAPHS, 1)), _whole((CLS, 64)),
            _whole((64, 64)), _whole((1, 64)),
            _whole((1, 64)), _whole((1, 64)), _whole((64, 64)), _whole((1, 64)),
            _whole((2, 64)), _whole((1, 64)), _whole((64, 64)), _whole((1, 64)),
            _whole((8, H)), _whole((1, H)), _whole((H, H)), _whole((1, H)),
            _whole((2 * (H + 192), H)), _whole((1, H)),
        ],
        out_specs=[_rows((BR, 192)), _rows((BR, H)), _rows((BR, H))],
        out_shape=[
            jax.ShapeDtypeStruct((NPAD, 192), f32),
            jax.ShapeDtypeStruct((NPAD, H), f32),
            jax.ShapeDtypeStruct((NPAD, H), f32),
        ],
    )(xt, geo_p, cat_p, bat_p, wall_batch, emb_table,
      cat_W, r2(cat_b), wall_W0, r2(wall_b0), wall_W1, r2(wall_b1),
      geo_W0, r2(geo_b0), geo_W1, r2(geo_b1), iW0, r2(init_b0), init_W1, r2(init_b1),
      m1_W0, r2(m1_b0))

    gride = (N_EDGES // BE,)

    def edge_call(Ad, Bs, W1, b1):
        return pl.pallas_call(
            _edge_body,
            grid=gride,
            in_specs=[_rows((BE, H)), _rows((BE, H)), _whole((H, H)), _whole((1, H))],
            out_specs=_rows((BE, H)),
            out_shape=jax.ShapeDtypeStruct((N_EDGES, H), f32),
        )(Ad, Bs, W1, b1)

    def mid_call(agg, cond, mW0, mb0):
        return pl.pallas_call(
            _node_mid_body,
            grid=gridn,
            in_specs=[_rows((BR, H)), _rows((BR, 192)),
                      _whole((2 * (H + 192), H)), _whole((1, H))],
            out_specs=[_rows((BR, H)), _rows((BR, H))],
            out_shape=[jax.ShapeDtypeStruct((NPAD, H), f32),
                       jax.ShapeDtypeStruct((NPAD, H), f32)],
        )(agg, cond, mW0, mb0)

    elist, llist, counts = _sc_prep(dstv)
    Ad, Bs = _sc_gather(A1, B1, dstv, srcv)
    msg1 = edge_call(Ad, Bs, m1_W1, r2(m1_b1))
    agg1 = _sc_scatter(msg1, elist, llist, counts).reshape(NPAD, H)
    A2, B2 = mid_call(agg1, cond, m2_W0, r2(m2_b0))
    Ad2, Bs2 = _sc_gather(A2, B2, dstv, srcv)
    msg2 = edge_call(Ad2, Bs2, m2_W1, r2(m2_b1))
    agg2 = _sc_scatter(msg2, elist, llist, counts).reshape(NPAD, H)

    out8 = pl.pallas_call(
        _tail_body,
        grid=gridn,
        in_specs=[_rows((BR, H)), _rows((BR, 192)), _rows((BR, 8)),
                  _whole((H + 192, H)), _whole((1, H)), _whole((H, 8)), _whole((1, 8))],
        out_specs=_rows((BR, 8)),
        out_shape=jax.ShapeDtypeStruct((NPAD, 8), f32),
    )(agg2, cond, tar8, tail_W0, r2(tail_b0), tW1, tb1)

    return out8[:N_NODES, :6]


# speedup vs baseline: 1.0051x; 1.0016x over previous
import jax
import jax.numpy as jnp
from jax import lax
from jax.experimental import pallas as pl
from jax.experimental.pallas import tpu as pltpu
from jax.experimental.pallas import tpu_sc as plsc

N_NODES = 10000
N_EDGES = 160000
N_GRAPHS = 512
CLS = 10
H = 128
LOG_STD_MIN, LOG_STD_MAX = -5.0, 2.0

NC, NS = 2, 16
NW = NC * NS
NPW = 320
NPAD = NW * NPW
EW = N_EDGES // NW
CG = 200
MB = 256
BR = 512
BE = 2000
NEG_INF = float("-inf")


def _node_stage0_body(xt_ref, geo_ref, cat_ref, bat_ref, wall_ref, emb_ref,
                      catW_ref, catb_ref,
                      wW0_ref, wb0_ref, wW1_ref, wb1_ref,
                      gW0_ref, gb0_ref, gW1_ref, gb1_ref,
                      iW0_ref, ib0_ref, iW1_ref, ib1_ref,
                      mW0_ref, mb0_ref,
                      cond_ref, A_ref, B_ref):
    f32 = jnp.float32
    emb_t = jnp.tanh(emb_ref[...])
    cat_oh = (cat_ref[...] == lax.broadcasted_iota(jnp.int32, (BR, CLS), 1)).astype(f32)
    class_feat = jnp.tanh((cat_oh @ emb_t) @ catW_ref[...] + catb_ref[...])
    ws = jnp.tanh(wall_ref[...] @ wW0_ref[...] + wb0_ref[...]) @ wW1_ref[...] + wb1_ref[...]
    ws = jnp.tanh(ws)
    bat_oh = (bat_ref[...] == lax.broadcasted_iota(jnp.int32, (BR, N_GRAPHS), 1)).astype(f32)
    wall_feat = bat_oh @ ws
    geo_feat = jnp.tanh(jnp.tanh(geo_ref[...] @ gW0_ref[...] + gb0_ref[...]) @ gW1_ref[...] + gb1_ref[...])
    obj = jnp.tanh(jnp.tanh(xt_ref[...] @ iW0_ref[...] + ib0_ref[...]) @ iW1_ref[...] + ib1_ref[...])
    cond = jnp.concatenate([class_feat, wall_feat, geo_feat], axis=1)
    W0 = mW0_ref[...]
    Wd = W0[: H + 192] - W0[H + 192:]
    Wb = W0[H + 192:]
    h0 = jnp.concatenate([obj, cond], axis=1)
    cond_ref[...] = cond
    A_ref[...] = h0 @ Wd + mb0_ref[...]
    B_ref[...] = h0 @ Wb


def _node_mid_body(agg_ref, cond_ref, mW0_ref, mb0_ref, A_ref, B_ref):
    agg = agg_ref[...].astype(jnp.float32)
    hh = jnp.tanh(jnp.where(jnp.isfinite(agg), agg, 0.0))
    W0 = mW0_ref[...]
    Wd = W0[: H + 192] - W0[H + 192:]
    Wb = W0[H + 192:]
    h = jnp.concatenate([hh, cond_ref[...]], axis=1)
    A_ref[...] = h @ Wd + mb0_ref[...]
    B_ref[...] = h @ Wb


def _edge_body(Ad_ref, Bs_ref, W1_ref, b1_ref, msg_ref):
    msg_ref[...] = jnp.tanh(Ad_ref[...] + Bs_ref[...]) @ W1_ref[...] + b1_ref[...]


def _tail_body(agg_ref, cond_ref, tar_ref, tW0_ref, tb0_ref, tW1_ref, tb1_ref, out_ref):
    agg = agg_ref[...].astype(jnp.float32)
    hh = jnp.tanh(jnp.where(jnp.isfinite(agg), agg, 0.0))
    h = jnp.concatenate([hh, cond_ref[...]], axis=1)
    t = jnp.tanh(h @ tW0_ref[...] + tb0_ref[...])
    o = t @ tW1_ref[...] + tb1_ref[...]
    lane = lax.broadcasted_iota(jnp.int32, (BR, 8), 1)
    to = jnp.tanh(o)
    mu = to + jnp.tanh(tar_ref[...])
    ls = LOG_STD_MIN + 0.5 * (LOG_STD_MAX - LOG_STD_MIN) * (to + 1.0)
    out_ref[...] = jnp.where(lane < 3, mu, jnp.exp(ls))


def _whole(shape):
    return pl.BlockSpec(shape, lambda i: (0,) * len(shape))


def _rows(bshape):
    return pl.BlockSpec(bshape, lambda i: (i,) + (0,) * (len(bshape) - 1))


def _sc_gather_body(A_hbm, B_hbm, dst_hbm, src_hbm, Ad_hbm, Bs_hbm,
                    didx0, sidx0, rowsA0, rowsB0, didx1, sidx1, rowsA1, rowsB1,
                    semA0, semB0, semA1, semB1, semSA0, semSB0, semSA1, semSB1):
    wid = lax.axis_index("s") * NC + lax.axis_index("c")
    base = wid * EW
    bufs = [(didx0, sidx0, rowsA0, rowsB0, semA0, semB0, semSA0, semSB0),
            (didx1, sidx1, rowsA1, rowsB1, semA1, semB1, semSA1, semSB1)]
    nch = EW // CG
    store_pend = [None, None]

    def stage(i, p):
        didx, sidx, rowsA, rowsB, semA, semB, _, _ = bufs[p]
        if store_pend[p] is not None:
            sa, sb = store_pend[p]
            sa.wait()
            sb.wait()
            store_pend[p] = None
        b = base + i * CG
        pltpu.sync_copy(dst_hbm.at[pl.ds(b, CG)], didx)
        pltpu.sync_copy(src_hbm.at[pl.ds(b, CG)], sidx)
        ca = pltpu.async_copy(A_hbm.at[didx], rowsA, semA)
        cb = pltpu.async_copy(B_hbm.at[sidx], rowsB, semB)
        return ca, cb

    pend = stage(0, 0)
    for i in range(nch):
        p = i % 2
        nxt = stage(i + 1, (i + 1) % 2) if i + 1 < nch else None
        ca, cb = pend
        ca.wait()
        cb.wait()
        _, _, rowsA, rowsB, _, _, semSA, semSB = bufs[p]
        b = base + i * CG
        sa = pltpu.async_copy(rowsA, Ad_hbm.at[pl.ds(b, CG)], semSA)
        sb = pltpu.async_copy(rowsB, Bs_hbm.at[pl.ds(b, CG)], semSB)
        store_pend[p] = (sa, sb)
        pend = nxt

    for p in range(2):
        if store_pend[p] is not None:
            sa, sb = store_pend[p]
            sa.wait()
            sb.wait()


def _sc_gather(A, B, dstv, srcv):
    mesh = plsc.VectorSubcoreMesh(core_axis_name="c", subcore_axis_name="s",
                                  num_cores=NC, num_subcores=NS)
    f = pl.kernel(
        _sc_gather_body,
        out_type=(jax.ShapeDtypeStruct((N_EDGES, H), jnp.float32),
                  jax.ShapeDtypeStruct((N_EDGES, H), jnp.float32)),
        mesh=mesh,
        scratch_types=[
            pltpu.VMEM((CG,), jnp.int32),
            pltpu.VMEM((CG,), jnp.int32),
            pltpu.VMEM((CG, H), jnp.float32),
            pltpu.VMEM((CG, H), jnp.float32),
            pltpu.VMEM((CG,), jnp.int32),
            pltpu.VMEM((CG,), jnp.int32),
            pltpu.VMEM((CG, H), jnp.float32),
            pltpu.VMEM((CG, H), jnp.float32),
            pltpu.SemaphoreType.DMA,
            pltpu.SemaphoreType.DMA,
            pltpu.SemaphoreType.DMA,
            pltpu.SemaphoreType.DMA,
            pltpu.SemaphoreType.DMA,
            pltpu.SemaphoreType.DMA,
            pltpu.SemaphoreType.DMA,
            pltpu.SemaphoreType.DMA,
        ],
        compiler_params=pltpu.CompilerParams(needs_layout_passes=False),
    )
    return f(A, B, dstv, srcv)


EPW = 668 * MB
DHALF = N_EDGES // 2


def _sc_prep_body(dst_hbm, elist_hbm, llist_hbm, cnt_hbm, dbuf, ebuf, lbuf, cbuf, sm):
    wid = lax.axis_index("s") * NC + lax.axis_index("c")
    lo = wid * NPW

    zero16 = jnp.zeros((16,), jnp.int32)
    dummy16 = jnp.full((16,), NPW, jnp.int32)

    def init_l(k, c):
        lbuf[pl.ds(k * 16, 16)] = dummy16
        return c

    def init_e(k, c):
        ebuf[pl.ds(k * 16, 16)] = zero16
        return c

    lax.fori_loop(0, MB // 16, init_l, 0)
    lax.fori_loop(0, MB // 16, init_e, 0)
    sm[0] = 0

    def flush():
        f = sm[0]
        base = wid * EPW + f * MB
        pltpu.sync_copy(ebuf, elist_hbm.at[pl.ds(base, MB)])
        pltpu.sync_copy(lbuf, llist_hbm.at[pl.ds(base, MB)])
        lax.fori_loop(0, MB // 16, init_l, 0)
        sm[0] = f + 1

    iota16 = lax.iota(jnp.int32, 16)
    npw_u = jnp.uint32(NPW)

    cnt_vec = zero16
    for hhalf in range(2):
        pltpu.sync_copy(dst_hbm.at[pl.ds(hhalf * DHALF, DHALF)], dbuf)

        def group(g, cv, hhalf=hhalf):
            d16 = dbuf[pl.ds(g * 16, 16)]
            dl = d16 - lo
            m = plsc.bitcast(dl, jnp.uint32) < npw_u
            eid = hhalf * DHALF + g * 16 + iota16
            n = cv[0]
            plsc.store_compressed(ebuf.at[pl.ds(n, 16)], eid, mask=m)
            plsc.store_compressed(lbuf.at[pl.ds(n, 16)], dl, mask=m)
            cv = cv + plsc.all_reduce_population_count(m)
            fp = cv[0] > MB - 16

            @pl.when(fp)
            def _():
                flush()

            return jnp.where(fp, zero16, cv)

        def group4(q, cv):
            for u in range(4):
                cv = group(q * 4 + u, cv)
            return cv

        cnt_vec = lax.fori_loop(0, DHALF // 64, group4, cnt_vec)

    @pl.when(cnt_vec[0] > 0)
    def _():
        flush()

    cbuf[pl.ds(0, 16)] = zero16 + sm[0]
    pltpu.sync_copy(cbuf, cnt_hbm.at[pl.ds(wid * 16, 16)])


def _sc_prep(dstv):
    mesh = plsc.VectorSubcoreMesh(core_axis_name="c", subcore_axis_name="s",
                                  num_cores=NC, num_subcores=NS)
    f = pl.kernel(
        _sc_prep_body,
        out_type=(jax.ShapeDtypeStruct((NW * EPW,), jnp.int32),
                  jax.ShapeDtypeStruct((NW * EPW,), jnp.int32),
                  jax.ShapeDtypeStruct((NW * 16,), jnp.int32)),
        mesh=mesh,
        scratch_types=[
            pltpu.VMEM((DHALF,), jnp.int32),
            pltpu.VMEM((MB,), jnp.int32),
            pltpu.VMEM((MB,), jnp.int32),
            pltpu.VMEM((16,), jnp.int32),
            pltpu.SMEM((2,), jnp.int32),
        ],
        compiler_params=pltpu.CompilerParams(needs_layout_passes=False),
    )
    return f(dstv)


def _sc_scatter_body(msg_hbm, elist_hbm, llist_hbm, cnt_hbm, agg_hbm,
                     acc, ebuf0, lbuf0, rows0, ebuf1, lbuf1, rows1, cbuf,
                     sem0, sem1, sem0b, sem1b):
    wid = lax.axis_index("s") * NC + lax.axis_index("c")
    lo = wid * NPW
    neg = jnp.full((16,), NEG_INF, jnp.float32)

    def init_acc(k, c):
        acc[pl.ds(k * 16, 16)] = neg
        return c

    lax.fori_loop(0, (NPW + 1) * H // 16, init_acc, 0)

    pltpu.sync_copy(cnt_hbm.at[pl.ds(wid * 16, 16)], cbuf)
    nf = cbuf[pl.ds(0, 16)][0]

    sets = [(ebuf0, lbuf0, rows0, sem0, sem0b), (ebuf1, lbuf1, rows1, sem1, sem1b)]
    HB = MB // 2

    def start(f, s):
        ebuf, lbuf, rows, sem, semb = s
        base = wid * EPW + f * MB
        pltpu.sync_copy(elist_hbm.at[pl.ds(base, MB)], ebuf)
        pltpu.sync_copy(llist_hbm.at[pl.ds(base, MB)], lbuf)
        pltpu.async_copy(msg_hbm.at[ebuf.at[pl.ds(0, HB)]], rows.at[pl.ds(0, HB)], sem)
        pltpu.async_copy(msg_hbm.at[ebuf.at[pl.ds(HB, HB)]], rows.at[pl.ds(HB, HB)], semb)

    def process(s):
        ebuf, lbuf, rows, sem, semb = s
        pltpu.make_async_copy(msg_hbm.at[ebuf.at[pl.ds(0, HB)]], rows.at[pl.ds(0, HB)], sem).wait()
        pltpu.make_async_copy(msg_hbm.at[ebuf.at[pl.ds(HB, HB)]], rows.at[pl.ds(HB, HB)], semb).wait()

        def one16(q, c2):
            dls = lbuf[pl.ds(q * 16, 16)]
            bases = [dls[t] * H for t in range(16)]
            for t in range(16):
                i = q * 16 + t
                for j in range(H // 16):
                    a = acc[pl.ds(bases[t] + j * 16, 16)]
                    r = rows[i, pl.ds(j * 16, 16)]
                    acc[pl.ds(bases[t] + j * 16, 16)] = jnp.maximum(a, r)
            return c2

        lax.fori_loop(0, MB // 16, one16, 0)

    @pl.when(nf > 0)
    def _():
        start(0, sets[0])

    def body(f, c):
        for par in range(2):
            @pl.when(f % 2 == par)
            def _(par=par):
                @pl.when(f + 1 < nf)
                def _():
                    start(f + 1, sets[1 - par])

                process(sets[par])

        return c

    lax.fori_loop(0, nf, body, 0)

    pltpu.sync_copy(acc.at[pl.ds(0, NPW * H)], agg_hbm.at[pl.ds(lo * H, NPW * H)])


def _sc_scatter(msg, elist, llist, counts):
    mesh = plsc.VectorSubcoreMesh(core_axis_name="c", subcore_axis_name="s",
                                  num_cores=NC, num_subcores=NS)
    f = pl.kernel(
        _sc_scatter_body,
        out_type=jax.ShapeDtypeStruct((NPAD * H,), jnp.float32),
        mesh=mesh,
        scratch_types=[
            pltpu.VMEM(((NPW + 1) * H,), jnp.float32),
            pltpu.VMEM((MB,), jnp.int32),
            pltpu.VMEM((MB,), jnp.int32),
            pltpu.VMEM((MB, H), jnp.float32),
            pltpu.VMEM((MB,), jnp.int32),
            pltpu.VMEM((MB,), jnp.int32),
            pltpu.VMEM((MB, H), jnp.float32),
            pltpu.VMEM((16,), jnp.int32),
            pltpu.SemaphoreType.DMA,
            pltpu.SemaphoreType.DMA,
            pltpu.SemaphoreType.DMA,
            pltpu.SemaphoreType.DMA,
        ],
        compiler_params=pltpu.CompilerParams(needs_layout_passes=False),
    )
    return f(msg, elist, llist, counts)


def kernel(x, geo, wall_batch, tar_scores, emb_table, cat_W, cat_b, wall_W0, wall_b0, wall_W1, wall_b1, geo_W0, geo_b0, geo_W1, geo_b1, init_W0, init_b0, init_W1, init_b1, m1_W0, m1_b0, m1_W1, m1_b1, m2_W0, m2_b0, m2_W1, m2_b1, tail_W0, tail_b0, tail_W1, tail_b1, category, edge_index, batch):
    f32 = jnp.float32

    def padn(a):
        return jnp.pad(a, ((0, NPAD - N_NODES), (0, 0)))

    def r2(b):
        return b[None, :]

    xt = jnp.pad(padn(jnp.concatenate([x, tar_scores], axis=1)), ((0, 0), (0, 1)))
    iW0 = jnp.pad(init_W0, ((0, 1), (0, 0)))
    geo_p = padn(geo)
    cat_p = padn(category)
    bat_p = padn(batch[:, None])
    tar8 = jnp.pad(padn(tar_scores), ((0, 0), (0, 5)))
    tW1 = jnp.pad(tail_W1, ((0, 0), (0, 2)))
    tb1 = jnp.pad(tail_b1, (0, 2))[None, :]
    srcv = edge_index[0]
    dstv = edge_index[1]

    gridn = (NPAD // BR,)
    cond, A1, B1 = pl.pallas_call(
        _node_stage0_body,
        grid=gridn,
        in_specs=[
            _rows((BR, 8)), _rows((BR, 2)), _rows((BR, 1)), _rows((BR, 1)),
            _whole((N_GRAPHS, 1)), _whole((CLS, 64)),
            _whole((64, 64)), _whole((1, 64)),
            _whole((1, 64)), _whole((1, 64)), _whole((64, 64)), _whole((1, 64)),
            _whole((2, 64)), _whole((1, 64)), _whole((64, 64)), _whole((1, 64)),
            _whole((8, H)), _whole((1, H)), _whole((H, H)), _whole((1, H)),
            _whole((2 * (H + 192), H)), _whole((1, H)),
        ],
        out_specs=[_rows((BR, 192)), _rows((BR, H)), _rows((BR, H))],
        out_shape=[
            jax.ShapeDtypeStruct((NPAD, 192), f32),
            jax.ShapeDtypeStruct((NPAD, H), f32),
            jax.ShapeDtypeStruct((NPAD, H), f32),
        ],
    )(xt, geo_p, cat_p, bat_p, wall_batch, emb_table,
      cat_W, r2(cat_b), wall_W0, r2(wall_b0), wall_W1, r2(wall_b1),
      geo_W0, r2(geo_b0), geo_W1, r2(geo_b1), iW0, r2(init_b0), init_W1, r2(init_b1),
      m1_W0, r2(m1_b0))

    gride = (N_EDGES // BE,)

    def edge_call(Ad, Bs, W1, b1):
        return pl.pallas_call(
            _edge_body,
            grid=gride,
            in_specs=[_rows((BE, H)), _rows((BE, H)), _whole((H, H)), _whole((1, H))],
            out_specs=_rows((BE, H)),
            out_shape=jax.ShapeDtypeStruct((N_EDGES, H), f32),
        )(Ad, Bs, W1, b1)

    def mid_call(agg, cond, mW0, mb0):
        return pl.pallas_call(
            _node_mid_body,
            grid=gridn,
            in_specs=[_rows((BR, H)), _rows((BR, 192)),
                      _whole((2 * (H + 192), H)), _whole((1, H))],
            out_specs=[_rows((BR, H)), _rows((BR, H))],
            out_shape=[jax.ShapeDtypeStruct((NPAD, H), f32),
                       jax.ShapeDtypeStruct((NPAD, H), f32)],
        )(agg, cond, mW0, mb0)

    elist, llist, counts = _sc_prep(dstv)
    Ad, Bs = _sc_gather(A1, B1, dstv, srcv)
    msg1 = edge_call(Ad, Bs, m1_W1, r2(m1_b1))
    agg1 = _sc_scatter(msg1, elist, llist, counts).reshape(NPAD, H)
    A2, B2 = mid_call(agg1, cond, m2_W0, r2(m2_b0))
    Ad2, Bs2 = _sc_gather(A2, B2, dstv, srcv)
    msg2 = edge_call(Ad2, Bs2, m2_W1, r2(m2_b1))
    agg2 = _sc_scatter(msg2, elist, llist, counts).reshape(NPAD, H)

    out8 = pl.pallas_call(
        _tail_body,
        grid=gridn,
        in_specs=[_rows((BR, H)), _rows((BR, 192)), _rows((BR, 8)),
                  _whole((H + 192, H)), _whole((1, H)), _whole((H, 8)), _whole((1, 8))],
        out_specs=_rows((BR, 8)),
        out_shape=jax.ShapeDtypeStruct((NPAD, 8), f32),
    )(agg2, cond, tar8, tail_W0, r2(tail_b0), tW1, tb1)

    return out8[:N_NODES, :6]
